# baseline (speedup 1.0000x reference)
import jax
import jax.numpy as jnp
from jax.experimental import pallas as pl
from jax.experimental.pallas import tpu as pltpu

N_ROWS = 100000
EMB = 256
HID = 128
OUT = 3
BLK = 7168


def _body(n_ref, c_ref, r_ref, w_ref, b_ref, wd_ref, bd_ref, out_ref):
    bc = jnp.dot(c_ref[...].astype(jnp.bfloat16), w_ref[:, 2 * HID:],
                 preferred_element_type=jnp.float32)
    nc = jnp.dot(n_ref[...].astype(jnp.bfloat16), w_ref[:, :HID],
                 preferred_element_type=jnp.float32)
    nc = jnp.maximum(nc + bc[:, :HID] + b_ref[:, :HID], 0.0)
    rc = jnp.dot(r_ref[...].astype(jnp.bfloat16), w_ref[:, HID:2 * HID],
                 preferred_element_type=jnp.float32)
    rc = jnp.maximum(rc + bc[:, HID:] + b_ref[:, HID:], 0.0)
    out = jnp.dot(nc.astype(jnp.bfloat16), wd_ref[:HID],
                  preferred_element_type=jnp.float32)
    out += jnp.dot(rc.astype(jnp.bfloat16), wd_ref[HID:],
                   preferred_element_type=jnp.float32)
    out_ref[...] = out + bd_ref[...]


def kernel(neighbor, current, remote, W_n, b_n, W_r, b_r, W_d, b_d):
    grid = (pl.cdiv(N_ROWS, BLK),)
    row_spec = pl.BlockSpec((BLK, EMB), lambda i: (i, 0))
    full = lambda shape: pl.BlockSpec(shape, lambda i: (0, 0))
    W_cat = jnp.concatenate(
        [W_n[:EMB], W_r[:EMB], W_n[EMB:], W_r[EMB:]],
        axis=1).astype(jnp.bfloat16)
    b_cat = jnp.concatenate([b_n, b_r]).reshape(1, 2 * HID)
    out = pl.pallas_call(
        _body,
        grid=grid,
        in_specs=[
            row_spec, row_spec, row_spec,
            full((EMB, 4 * HID)), full((1, 2 * HID)),
            full((2 * HID, OUT)), full((1, OUT)),
        ],
        out_specs=pl.BlockSpec((BLK, OUT), lambda i: (i, 0)),
        out_shape=jax.ShapeDtypeStruct((N_ROWS, OUT), jnp.float32),
        compiler_params=pltpu.CompilerParams(
            dimension_semantics=(pltpu.ARBITRARY,),
            vmem_limit_bytes=100 * 1024 * 1024),
    )(neighbor, current, remote, W_cat, b_cat,
      W_d.astype(jnp.bfloat16), b_d.reshape(1, OUT))
    return out


# speedup vs baseline: 1.0006x; 1.0006x over previous
import jax
import jax.numpy as jnp
from jax.experimental import pallas as pl
from jax.experimental.pallas import tpu as pltpu

N_ROWS = 100000
EMB = 256
HID = 128
OUT = 3
BLK = 7168


def _body(n_ref, c_ref, r_ref, w_ref, b_ref, wd_ref, bd_ref, out_ref):
    bc = jnp.dot(c_ref[...].astype(jnp.bfloat16), w_ref[:, 2 * HID:],
                 preferred_element_type=jnp.float32)
    nc = jnp.dot(n_ref[...].astype(jnp.bfloat16), w_ref[:, :HID],
                 preferred_element_type=jnp.float32)
    nc = jnp.maximum(nc + bc[:, :HID] + b_ref[:, :HID], 0.0)
    rc = jnp.dot(r_ref[...].astype(jnp.bfloat16), w_ref[:, HID:2 * HID],
                 preferred_element_type=jnp.float32)
    rc = jnp.maximum(rc + bc[:, HID:] + b_ref[:, HID:], 0.0)
    out = jnp.dot(nc.astype(jnp.bfloat16), wd_ref[:HID],
                  preferred_element_type=jnp.float32)
    out += jnp.dot(rc.astype(jnp.bfloat16), wd_ref[HID:],
                   preferred_element_type=jnp.float32)
    out_ref[...] = out + bd_ref[...]


def kernel(neighbor, current, remote, W_n, b_n, W_r, b_r, W_d, b_d):
    grid = (pl.cdiv(N_ROWS, BLK),)
    row_spec = pl.BlockSpec((BLK, EMB), lambda i: (i, 0))
    full = lambda shape: pl.BlockSpec(shape, lambda i: (0, 0))
    W_cat = jnp.concatenate(
        [W_n[:EMB], W_r[:EMB], W_n[EMB:], W_r[EMB:]],
        axis=1).astype(jnp.bfloat16)
    b_cat = jnp.concatenate([b_n, b_r]).reshape(1, 2 * HID)
    out = pl.pallas_call(
        _body,
        grid=grid,
        in_specs=[
            row_spec, row_spec, row_spec,
            full((EMB, 4 * HID)), full((1, 2 * HID)),
            full((2 * HID, OUT)), full((1, OUT)),
        ],
        out_specs=pl.BlockSpec((BLK, OUT), lambda i: (i, 0)),
        out_shape=jax.ShapeDtypeStruct((N_ROWS, OUT), jnp.float32),
        compiler_params=pltpu.CompilerParams(
            dimension_semantics=(pltpu.PARALLEL,),
            vmem_limit_bytes=100 * 1024 * 1024),
    )(neighbor, current, remote, W_cat, b_cat,
      W_d.astype(jnp.bfloat16), b_d.reshape(1, OUT))
    return out
